# baseline (speedup 1.0000x reference)
import functools

import jax
import jax.numpy as jnp
from jax import lax
from jax.experimental import pallas as pl
from jax.experimental.pallas import tpu as pltpu
from jax.experimental.pallas import tpu_sc as plsc

NC = 2
NS = 16
NW = NC * NS
CH = 128
NBUF = 2
NNODE = 10000
FDIM = 128
N_PAD = 10240
RPS = N_PAD // NS


def _seg_loop(with_cnt, nch, feat, wid, src3, dst_v, acc_sh, cnt_sh, ones_v,
              isrc, rows, isem, gsem, ssem, csem):
  ng = nch // NBUF

  def _idx(j, b):
    return pltpu.make_async_copy(src3.at[wid, j], isrc[b], isem.at[b])

  def _gather(b):
    return pltpu.make_async_copy(feat.at[isrc[b]], rows[b], gsem.at[b])

  def _scat(j, b):
    return pltpu.async_copy(rows[b], acc_sh.at[dst_v.at[j]], ssem.at[b],
                            add=True)

  def _cnt(j, b):
    return pltpu.async_copy(ones_v, cnt_sh.at[dst_v.at[j]], csem.at[b],
                            add=True)

  for b in range(NBUF):
    _idx(b, b).start()
    _idx(b, b).wait()
    _gather(b).start()

  def group(t, carry):
    base = t * NBUF
    descs = []
    for b in range(NBUF):
      j = base + b
      _gather(b).wait()
      sd = _scat(j, b)
      cd = _cnt(j, b) if with_cnt else None
      descs.append((sd, cd))
      _idx((j + NBUF) % nch, b).start()
    for b in range(NBUF):
      j = base + b
      sd, cd = descs[b]
      sd.wait()
      if cd is not None:
        cd.wait()
      _idx(j, b).wait()
      _gather(b).start()
    return carry

  lax.fori_loop(0, ng, group, 0)
  for b in range(NBUF):
    _gather(b).wait()


def _seg_sum_cnt_body(feat, src3, dst3, z2d, z1d, ones_h, out_p, out_c,
                      acc_sh, cnt_sh, dst_v, ones_v,
                      isrc0, isrc1, rows0, rows1, isem, gsem, ssem, csem,
                      *, nch):
  c = lax.axis_index("c")
  s = lax.axis_index("s")
  wid = s * NC + c
  base = s * RPS
  pltpu.sync_copy(z2d.at[pl.ds(base, RPS)], acc_sh.at[pl.ds(base, RPS)])
  pltpu.sync_copy(z1d.at[pl.ds(base, RPS)], cnt_sh.at[pl.ds(base, RPS)])
  pltpu.sync_copy(ones_h, ones_v)
  pltpu.sync_copy(dst3.at[wid], dst_v)
  plsc.subcore_barrier()
  _seg_loop(True, nch, feat, wid, src3, dst_v, acc_sh, cnt_sh, ones_v,
            (isrc0, isrc1), (rows0, rows1), isem, gsem, ssem, csem)
  plsc.subcore_barrier()
  pltpu.sync_copy(acc_sh.at[pl.ds(base, RPS)], out_p.at[c, pl.ds(base, RPS)])
  pltpu.sync_copy(cnt_sh.at[pl.ds(base, RPS)], out_c.at[c, pl.ds(base, RPS)])


def _seg_sum_body(feat, src3, dst3, z2d, out_p,
                  acc_sh, dst_v,
                  isrc0, isrc1, rows0, rows1, isem, gsem, ssem,
                  *, nch):
  c = lax.axis_index("c")
  s = lax.axis_index("s")
  wid = s * NC + c
  base = s * RPS
  pltpu.sync_copy(z2d.at[pl.ds(base, RPS)], acc_sh.at[pl.ds(base, RPS)])
  pltpu.sync_copy(dst3.at[wid], dst_v)
  plsc.subcore_barrier()
  _seg_loop(False, nch, feat, wid, src3, dst_v, acc_sh, None, None,
            (isrc0, isrc1), (rows0, rows1), isem, gsem, ssem, None)
  plsc.subcore_barrier()
  pltpu.sync_copy(acc_sh.at[pl.ds(base, RPS)], out_p.at[c, pl.ds(base, RPS)])


def _make_seg_kernels(nch):
  mesh = plsc.VectorSubcoreMesh(core_axis_name="c", subcore_axis_name="s")
  ring_bufs = [pltpu.VMEM((CH,), jnp.int32) for _ in range(NBUF)] + [
      pltpu.VMEM((CH, FDIM), jnp.float32) for _ in range(NBUF)]
  seg_cnt = pl.kernel(
      functools.partial(_seg_sum_cnt_body, nch=nch),
      out_type=(jax.ShapeDtypeStruct((NC, N_PAD, FDIM), jnp.float32),
                jax.ShapeDtypeStruct((NC, N_PAD), jnp.float32)),
      mesh=mesh,
      scratch_types=[
          pltpu.VMEM_SHARED((N_PAD, FDIM), jnp.float32),
          pltpu.VMEM_SHARED((N_PAD,), jnp.float32),
          pltpu.VMEM((nch, CH), jnp.int32),
          pltpu.VMEM((CH,), jnp.float32),
      ] + ring_bufs + [
          pltpu.SemaphoreType.DMA((NBUF,)),
          pltpu.SemaphoreType.DMA((NBUF,)),
          pltpu.SemaphoreType.DMA((NBUF,)),
          pltpu.SemaphoreType.DMA((NBUF,)),
      ],
      name="sage_seg_sum_cnt",
  )
  seg = pl.kernel(
      functools.partial(_seg_sum_body, nch=nch),
      out_type=jax.ShapeDtypeStruct((NC, N_PAD, FDIM), jnp.float32),
      mesh=mesh,
      scratch_types=[
          pltpu.VMEM_SHARED((N_PAD, FDIM), jnp.float32),
          pltpu.VMEM((nch, CH), jnp.int32),
      ] + ring_bufs + [
          pltpu.SemaphoreType.DMA((NBUF,)),
          pltpu.SemaphoreType.DMA((NBUF,)),
          pltpu.SemaphoreType.DMA((NBUF,)),
      ],
      name="sage_seg_sum",
  )
  return seg_cnt, seg


BR = 1000


def _dense_body(x_ref, p_ref, c_ref, ws_ref, wn_ref, b_ref, o_ref):
  p = p_ref[0] + p_ref[1]
  cnt = c_ref[0] + c_ref[1]
  agg = p / jnp.maximum(cnt, 1.0)
  acc = jnp.dot(x_ref[...], ws_ref[...], preferred_element_type=jnp.float32)
  acc = acc + jnp.dot(agg, wn_ref[...], preferred_element_type=jnp.float32)
  o_ref[...] = jnp.maximum(acc + b_ref[...], 0.0)


def _dense(x, p, cnt3, ws, wn, b):
  nb = NNODE // BR
  return pl.pallas_call(
      _dense_body,
      grid=(nb,),
      in_specs=[
          pl.BlockSpec((BR, FDIM), lambda i: (i, 0)),
          pl.BlockSpec((NC, BR, FDIM), lambda i: (0, i, 0)),
          pl.BlockSpec((NC, BR, 1), lambda i: (0, i, 0)),
          pl.BlockSpec((FDIM, FDIM), lambda i: (0, 0)),
          pl.BlockSpec((FDIM, FDIM), lambda i: (0, 0)),
          pl.BlockSpec((1, FDIM), lambda i: (0, 0)),
      ],
      out_specs=pl.BlockSpec((BR, FDIM), lambda i: (i, 0)),
      out_shape=jax.ShapeDtypeStruct((NNODE, FDIM), jnp.float32),
  )(x, p, cnt3, ws, wn, b.reshape(1, FDIM))


def kernel(x, edge_index, W_self1, W_neigh1, b1, W_self2, W_neigh2, b2):
  e = edge_index.shape[1]
  nch = -(-e // (NW * CH))
  nch = -(-nch // NBUF) * NBUF
  e_pad = NW * nch * CH
  src = edge_index[0]
  dst = edge_index[1]
  pad = e_pad - e
  src3 = jnp.concatenate(
      [src, jnp.zeros((pad,), jnp.int32)]).reshape(NW, nch, CH)
  dst3 = jnp.concatenate(
      [dst, jnp.full((pad,), NNODE, jnp.int32)]).reshape(NW, nch, CH)
  z2d = jnp.zeros((N_PAD, FDIM), jnp.float32)
  z1d = jnp.zeros((N_PAD,), jnp.float32)
  ones_h = jnp.ones((CH,), jnp.float32)

  seg_cnt, seg = _make_seg_kernels(nch)
  p1, cnts = seg_cnt(x, src3, dst3, z2d, z1d, ones_h)
  cnt3 = cnts.reshape(NC, N_PAD, 1)
  h = _dense(x, p1, cnt3, W_self1, W_neigh1, b1)
  p2 = seg(h, src3, dst3, z2d)
  return _dense(h, p2, cnt3, W_self2, W_neigh2, b2)


# speedup vs baseline: 1.3693x; 1.3693x over previous
import functools

import jax
import jax.numpy as jnp
from jax import lax
from jax.experimental import pallas as pl
from jax.experimental.pallas import tpu as pltpu
from jax.experimental.pallas import tpu_sc as plsc

NC = 2
NS = 16
NW = NC * NS
CH = 128
NBUF = 2
NNODE = 10000
FDIM = 128
N_PAD = 10240
RPS = N_PAD // NS

Q0 = 34
Q1 = 124


def _seg_loop(with_cnt, q, feat, wid, src3, dst3, acc_sh, cnt_sh, ones_v,
              isrc, idst, rows, isem, dsem, gsem, ssem, csem):
  ng = q // NBUF

  def _isrc(j, b):
    return pltpu.make_async_copy(src3.at[wid, j], isrc[b], isem.at[b])

  def _idst(j, b):
    return pltpu.make_async_copy(dst3.at[wid, pl.ds(j, 1)], idst[b],
                                 dsem.at[b])

  def _gather(b):
    return pltpu.make_async_copy(feat.at[isrc[b]], rows[b], gsem.at[b])

  def _scat(b):
    return pltpu.async_copy(rows[b], acc_sh.at[idst[b].at[0]], ssem.at[b],
                            add=True)

  def _cnt(b):
    return pltpu.async_copy(ones_v, cnt_sh.at[idst[b].at[0]], csem.at[b],
                            add=True)

  for b in range(NBUF):
    _isrc(b, b).start()
    _idst(b, b).start()
    _isrc(b, b).wait()
    _gather(b).start()

  def group(t, carry):
    base = t * NBUF
    descs = []
    for b in range(NBUF):
      j = base + b
      _gather(b).wait()
      _idst(j, b).wait()
      sd = _scat(b)
      cd = _cnt(b) if with_cnt else None
      descs.append((sd, cd))
      _isrc((j + NBUF) % q, b).start()
    for b in range(NBUF):
      j = base + b
      sd, cd = descs[b]
      sd.wait()
      if cd is not None:
        cd.wait()
      _idst((j + NBUF) % q, b).start()
      _isrc(j, b).wait()
      _gather(b).start()
    return carry

  lax.fori_loop(0, ng, group, 0)
  for b in range(NBUF):
    _gather(b).wait()
    _idst(b, b).wait()


def _seg_sum_cnt_body(feat, src3, dst3, z2d, z1d, ones_h, out_p, out_c,
                      acc_sh, cnt_sh, ones_v,
                      isrc0, isrc1, idst0, idst1, rows0, rows1,
                      isem, dsem, gsem, ssem, csem):
  c = lax.axis_index("c")
  s = lax.axis_index("s")
  wid = c * NS + s
  q = jnp.where(c == 0, Q0, Q1)
  base = s * RPS
  pltpu.sync_copy(z2d.at[pl.ds(base, RPS)], acc_sh.at[pl.ds(base, RPS)])
  pltpu.sync_copy(z1d.at[pl.ds(base, RPS)], cnt_sh.at[pl.ds(base, RPS)])
  pltpu.sync_copy(ones_h, ones_v)
  plsc.subcore_barrier()
  _seg_loop(True, q, feat, wid, src3, dst3, acc_sh, cnt_sh, ones_v,
            (isrc0, isrc1), (idst0, idst1), (rows0, rows1),
            isem, dsem, gsem, ssem, csem)
  plsc.subcore_barrier()
  pltpu.sync_copy(acc_sh.at[pl.ds(base, RPS)], out_p.at[c, pl.ds(base, RPS)])
  pltpu.sync_copy(cnt_sh.at[pl.ds(base, RPS)], out_c.at[c, pl.ds(base, RPS)])


def _seg_sum_body(feat, src3, dst3, z2d, out_p,
                  acc_sh,
                  isrc0, isrc1, idst0, idst1, rows0, rows1,
                  isem, dsem, gsem, ssem):
  c = lax.axis_index("c")
  s = lax.axis_index("s")
  wid = c * NS + s
  q = jnp.where(c == 0, Q0, Q1)
  base = s * RPS
  pltpu.sync_copy(z2d.at[pl.ds(base, RPS)], acc_sh.at[pl.ds(base, RPS)])
  plsc.subcore_barrier()
  _seg_loop(False, q, feat, wid, src3, dst3, acc_sh, None, None,
            (isrc0, isrc1), (idst0, idst1), (rows0, rows1),
            isem, dsem, gsem, ssem, None)
  plsc.subcore_barrier()
  pltpu.sync_copy(acc_sh.at[pl.ds(base, RPS)], out_p.at[c, pl.ds(base, RPS)])


def _make_seg_kernels():
  mesh = plsc.VectorSubcoreMesh(core_axis_name="c", subcore_axis_name="s")
  ring_bufs = (
      [pltpu.VMEM((CH,), jnp.int32) for _ in range(NBUF)] +
      [pltpu.VMEM((1, CH), jnp.int32) for _ in range(NBUF)] +
      [pltpu.VMEM((CH, FDIM), jnp.float32) for _ in range(NBUF)])
  seg_cnt = pl.kernel(
      _seg_sum_cnt_body,
      out_type=(jax.ShapeDtypeStruct((NC, N_PAD, FDIM), jnp.float32),
                jax.ShapeDtypeStruct((NC, N_PAD), jnp.float32)),
      mesh=mesh,
      scratch_types=[
          pltpu.VMEM_SHARED((N_PAD, FDIM), jnp.float32),
          pltpu.VMEM_SHARED((N_PAD,), jnp.float32),
          pltpu.VMEM((CH,), jnp.float32),
      ] + ring_bufs + [
          pltpu.SemaphoreType.DMA((NBUF,)),
          pltpu.SemaphoreType.DMA((NBUF,)),
          pltpu.SemaphoreType.DMA((NBUF,)),
          pltpu.SemaphoreType.DMA((NBUF,)),
          pltpu.SemaphoreType.DMA((NBUF,)),
      ],
      name="sage_seg_sum_cnt",
  )
  seg = pl.kernel(
      _seg_sum_body,
      out_type=jax.ShapeDtypeStruct((NC, N_PAD, FDIM), jnp.float32),
      mesh=mesh,
      scratch_types=[
          pltpu.VMEM_SHARED((N_PAD, FDIM), jnp.float32),
      ] + ring_bufs + [
          pltpu.SemaphoreType.DMA((NBUF,)),
          pltpu.SemaphoreType.DMA((NBUF,)),
          pltpu.SemaphoreType.DMA((NBUF,)),
          pltpu.SemaphoreType.DMA((NBUF,)),
      ],
      name="sage_seg_sum",
  )
  return seg_cnt, seg


BR = 1000


def _dense_body(x_ref, p_ref, c_ref, ws_ref, wn_ref, b_ref, o_ref):
  p = p_ref[0] + p_ref[1]
  cnt = c_ref[0] + c_ref[1]
  agg = p / jnp.maximum(cnt, 1.0)
  acc = jnp.dot(x_ref[...], ws_ref[...], preferred_element_type=jnp.float32)
  acc = acc + jnp.dot(agg, wn_ref[...], preferred_element_type=jnp.float32)
  o_ref[...] = jnp.maximum(acc + b_ref[...], 0.0)


def _dense(x, p, cnt3, ws, wn, b):
  nb = NNODE // BR
  return pl.pallas_call(
      _dense_body,
      grid=(nb,),
      in_specs=[
          pl.BlockSpec((BR, FDIM), lambda i: (i, 0)),
          pl.BlockSpec((NC, BR, FDIM), lambda i: (0, i, 0)),
          pl.BlockSpec((NC, BR, 1), lambda i: (0, i, 0)),
          pl.BlockSpec((FDIM, FDIM), lambda i: (0, 0)),
          pl.BlockSpec((FDIM, FDIM), lambda i: (0, 0)),
          pl.BlockSpec((1, FDIM), lambda i: (0, 0)),
      ],
      out_specs=pl.BlockSpec((BR, FDIM), lambda i: (i, 0)),
      out_shape=jax.ShapeDtypeStruct((NNODE, FDIM), jnp.float32),
  )(x, p, cnt3, ws, wn, b.reshape(1, FDIM))


def _chunk_layout(idx, fill):
  e = idx.shape[0]
  qmax = max(Q0, Q1)
  a = NS * Q0 * CH
  e_pad = NS * (Q0 + Q1) * CH
  idx = jnp.concatenate([idx, jnp.full((e_pad - e,), fill, jnp.int32)])
  part0 = idx[:a].reshape(NS, Q0, CH)
  part1 = idx[a:].reshape(NS, Q1, CH)
  part0 = jnp.pad(part0, ((0, 0), (0, qmax - Q0), (0, 0)),
                  constant_values=fill)
  part1 = jnp.pad(part1, ((0, 0), (0, qmax - Q1), (0, 0)),
                  constant_values=fill)
  return jnp.concatenate([part0, part1], axis=0)


def kernel(x, edge_index, W_self1, W_neigh1, b1, W_self2, W_neigh2, b2):
  src3 = _chunk_layout(edge_index[0], 0)
  dst3 = _chunk_layout(edge_index[1], NNODE)
  z2d = jnp.zeros((N_PAD, FDIM), jnp.float32)
  z1d = jnp.zeros((N_PAD,), jnp.float32)
  ones_h = jnp.ones((CH,), jnp.float32)

  seg_cnt, seg = _make_seg_kernels()
  p1, cnts = seg_cnt(x, src3, dst3, z2d, z1d, ones_h)
  cnt3 = cnts.reshape(NC, N_PAD, 1)
  h = _dense(x, p1, cnt3, W_self1, W_neigh1, b1)
  p2 = seg(h, src3, dst3, z2d)
  return _dense(h, p2, cnt3, W_self2, W_neigh2, b2)


# speedup vs baseline: 1.6178x; 1.1815x over previous
import functools

import jax
import jax.numpy as jnp
from jax import lax
from jax.experimental import pallas as pl
from jax.experimental.pallas import tpu as pltpu
from jax.experimental.pallas import tpu_sc as plsc

NC = 2
NS = 16
NW = NC * NS
CH = 128
NBUF = 2
NNODE = 10000
FDIM = 128
N_PAD = 10240
RPS = N_PAD // NS

Q0 = 80
Q1 = 78


def _seg_loop(with_cnt, q, feat, wid, src3, dst3, acc_sh, cnt_sh, ones_v,
              isrc, idst, rows, isem, dsem, gsem, ssem, csem):
  ng = q // NBUF

  def _isrc(j, b):
    return pltpu.make_async_copy(src3.at[wid, j], isrc[b], isem.at[b])

  def _idst(j, b):
    return pltpu.make_async_copy(dst3.at[wid, pl.ds(j, 1)], idst[b],
                                 dsem.at[b])

  def _gather(b):
    return pltpu.make_async_copy(feat.at[isrc[b]], rows[b], gsem.at[b])

  def _scat(b):
    return pltpu.async_copy(rows[b], acc_sh.at[idst[b].at[0]], ssem.at[b],
                            add=True)

  def _cnt(b):
    return pltpu.async_copy(ones_v, cnt_sh.at[idst[b].at[0]], csem.at[b],
                            add=True)

  for b in range(NBUF):
    _isrc(b, b).start()
    _idst(b, b).start()
    _isrc(b, b).wait()
    _gather(b).start()

  def group(t, carry):
    base = t * NBUF
    descs = []
    for b in range(NBUF):
      j = base + b
      _gather(b).wait()
      _idst(j, b).wait()
      sd = _scat(b)
      cd = _cnt(b) if with_cnt else None
      descs.append((sd, cd))
      _isrc((j + NBUF) % q, b).start()
    for b in range(NBUF):
      j = base + b
      sd, cd = descs[b]
      sd.wait()
      if cd is not None:
        cd.wait()
      _idst((j + NBUF) % q, b).start()
      _isrc(j, b).wait()
      _gather(b).start()
    return carry

  lax.fori_loop(0, ng, group, 0)
  for b in range(NBUF):
    _gather(b).wait()
    _idst(b, b).wait()


def _seg_sum_cnt_body(feat, src3, dst3, z2d, z1d, ones_h, out_p, out_c,
                      acc_sh, cnt_sh, ones_v,
                      isrc0, isrc1, idst0, idst1, rows0, rows1,
                      isem, dsem, gsem, ssem, csem):
  c = lax.axis_index("c")
  s = lax.axis_index("s")
  wid = c * NS + s
  q = jnp.where(c == 0, Q0, Q1)
  base = s * RPS
  pltpu.sync_copy(z2d.at[pl.ds(base, RPS)], acc_sh.at[pl.ds(base, RPS)])
  pltpu.sync_copy(z1d.at[pl.ds(base, RPS)], cnt_sh.at[pl.ds(base, RPS)])
  pltpu.sync_copy(ones_h, ones_v)
  plsc.subcore_barrier()
  _seg_loop(True, q, feat, wid, src3, dst3, acc_sh, cnt_sh, ones_v,
            (isrc0, isrc1), (idst0, idst1), (rows0, rows1),
            isem, dsem, gsem, ssem, csem)
  plsc.subcore_barrier()
  pltpu.sync_copy(acc_sh.at[pl.ds(base, RPS)], out_p.at[c, pl.ds(base, RPS)])
  pltpu.sync_copy(cnt_sh.at[pl.ds(base, RPS)], out_c.at[c, pl.ds(base, RPS)])


def _seg_sum_body(feat, src3, dst3, z2d, out_p,
                  acc_sh,
                  isrc0, isrc1, idst0, idst1, rows0, rows1,
                  isem, dsem, gsem, ssem):
  c = lax.axis_index("c")
  s = lax.axis_index("s")
  wid = c * NS + s
  q = jnp.where(c == 0, Q0, Q1)
  base = s * RPS
  pltpu.sync_copy(z2d.at[pl.ds(base, RPS)], acc_sh.at[pl.ds(base, RPS)])
  plsc.subcore_barrier()
  _seg_loop(False, q, feat, wid, src3, dst3, acc_sh, None, None,
            (isrc0, isrc1), (idst0, idst1), (rows0, rows1),
            isem, dsem, gsem, ssem, None)
  plsc.subcore_barrier()
  pltpu.sync_copy(acc_sh.at[pl.ds(base, RPS)], out_p.at[c, pl.ds(base, RPS)])


def _make_seg_kernels():
  mesh = plsc.VectorSubcoreMesh(core_axis_name="c", subcore_axis_name="s")
  ring_bufs = (
      [pltpu.VMEM((CH,), jnp.int32) for _ in range(NBUF)] +
      [pltpu.VMEM((1, CH), jnp.int32) for _ in range(NBUF)] +
      [pltpu.VMEM((CH, FDIM), jnp.float32) for _ in range(NBUF)])
  seg_cnt = pl.kernel(
      _seg_sum_cnt_body,
      out_type=(jax.ShapeDtypeStruct((NC, N_PAD, FDIM), jnp.float32),
                jax.ShapeDtypeStruct((NC, N_PAD), jnp.float32)),
      mesh=mesh,
      scratch_types=[
          pltpu.VMEM_SHARED((N_PAD, FDIM), jnp.float32),
          pltpu.VMEM_SHARED((N_PAD,), jnp.float32),
          pltpu.VMEM((CH,), jnp.float32),
      ] + ring_bufs + [
          pltpu.SemaphoreType.DMA((NBUF,)),
          pltpu.SemaphoreType.DMA((NBUF,)),
          pltpu.SemaphoreType.DMA((NBUF,)),
          pltpu.SemaphoreType.DMA((NBUF,)),
          pltpu.SemaphoreType.DMA((NBUF,)),
      ],
      name="sage_seg_sum_cnt",
  )
  seg = pl.kernel(
      _seg_sum_body,
      out_type=jax.ShapeDtypeStruct((NC, N_PAD, FDIM), jnp.float32),
      mesh=mesh,
      scratch_types=[
          pltpu.VMEM_SHARED((N_PAD, FDIM), jnp.float32),
      ] + ring_bufs + [
          pltpu.SemaphoreType.DMA((NBUF,)),
          pltpu.SemaphoreType.DMA((NBUF,)),
          pltpu.SemaphoreType.DMA((NBUF,)),
          pltpu.SemaphoreType.DMA((NBUF,)),
      ],
      name="sage_seg_sum",
  )
  return seg_cnt, seg


BR = 1000


def _dense_body(x_ref, p_ref, c_ref, ws_ref, wn_ref, b_ref, o_ref):
  p = p_ref[0] + p_ref[1]
  cnt = c_ref[0] + c_ref[1]
  agg = p / jnp.maximum(cnt, 1.0)
  acc = jnp.dot(x_ref[...], ws_ref[...], preferred_element_type=jnp.float32)
  acc = acc + jnp.dot(agg, wn_ref[...], preferred_element_type=jnp.float32)
  o_ref[...] = jnp.maximum(acc + b_ref[...], 0.0)


def _dense(x, p, cnt3, ws, wn, b):
  nb = NNODE // BR
  return pl.pallas_call(
      _dense_body,
      grid=(nb,),
      in_specs=[
          pl.BlockSpec((BR, FDIM), lambda i: (i, 0)),
          pl.BlockSpec((NC, BR, FDIM), lambda i: (0, i, 0)),
          pl.BlockSpec((NC, BR, 1), lambda i: (0, i, 0)),
          pl.BlockSpec((FDIM, FDIM), lambda i: (0, 0)),
          pl.BlockSpec((FDIM, FDIM), lambda i: (0, 0)),
          pl.BlockSpec((1, FDIM), lambda i: (0, 0)),
      ],
      out_specs=pl.BlockSpec((BR, FDIM), lambda i: (i, 0)),
      out_shape=jax.ShapeDtypeStruct((NNODE, FDIM), jnp.float32),
  )(x, p, cnt3, ws, wn, b.reshape(1, FDIM))


def _chunk_layout(idx, fill):
  e = idx.shape[0]
  qmax = max(Q0, Q1)
  a = NS * Q0 * CH
  e_pad = NS * (Q0 + Q1) * CH
  idx = jnp.concatenate([idx, jnp.full((e_pad - e,), fill, jnp.int32)])
  part0 = idx[:a].reshape(NS, Q0, CH)
  part1 = idx[a:].reshape(NS, Q1, CH)
  part0 = jnp.pad(part0, ((0, 0), (0, qmax - Q0), (0, 0)),
                  constant_values=fill)
  part1 = jnp.pad(part1, ((0, 0), (0, qmax - Q1), (0, 0)),
                  constant_values=fill)
  return jnp.concatenate([part0, part1], axis=0)


def kernel(x, edge_index, W_self1, W_neigh1, b1, W_self2, W_neigh2, b2):
  src3 = _chunk_layout(edge_index[0], 0)
  dst3 = _chunk_layout(edge_index[1], NNODE)
  z2d = jnp.zeros((N_PAD, FDIM), jnp.float32)
  z1d = jnp.zeros((N_PAD,), jnp.float32)
  ones_h = jnp.ones((CH,), jnp.float32)

  seg_cnt, seg = _make_seg_kernels()
  p1, cnts = seg_cnt(x, src3, dst3, z2d, z1d, ones_h)
  cnt3 = cnts.reshape(NC, N_PAD, 1)
  h = _dense(x, p1, cnt3, W_self1, W_neigh1, b1)
  p2 = seg(h, src3, dst3, z2d)
  return _dense(h, p2, cnt3, W_self2, W_neigh2, b2)


# speedup vs baseline: 3.1498x; 1.9470x over previous
import jax
import jax.numpy as jnp
from jax import lax
from jax.experimental import pallas as pl
from jax.experimental.pallas import tpu as pltpu
from jax.experimental.pallas import tpu_sc as plsc

NC = 2
NS = 16
NW = NC * NS
CH = 128
NBUF = 2
NNODE = 10000
FDIM = 128
N_PAD = 10112
RPS = N_PAD // NS

NCHUNK = -(-320000 // CH)
QBASE = NCHUNK // NW
QREM = NCHUNK % NW


def _seg_loop(with_cnt, q, off_e, feat, edges, acc_sh, cnt_sh, ones_v,
              isrc, idst, rows, isem, dsem, gsem, ssem, csem):
  ng = q // NBUF

  def _isrc(j, b):
    return pltpu.make_async_copy(
        edges.at[0, pl.ds(off_e + j * CH, CH)], isrc[b], isem.at[b])

  def _idst(j, b):
    return pltpu.make_async_copy(
        edges.at[pl.ds(1, 1), pl.ds(off_e + j * CH, CH)], idst[b],
        dsem.at[b])

  def _gather(b):
    return pltpu.make_async_copy(feat.at[isrc[b]], rows[b], gsem.at[b])

  def _scat(b):
    return pltpu.async_copy(rows[b], acc_sh.at[idst[b].at[0]], ssem.at[b],
                            add=True)

  def _cnt(b):
    return pltpu.async_copy(ones_v, cnt_sh.at[idst[b].at[0]], csem.at[b],
                            add=True)

  for b in range(NBUF):
    _isrc(b, b).start()
    _idst(b, b).start()
    _isrc(b, b).wait()
    _gather(b).start()

  def group(t, carry):
    base = t * NBUF
    descs = []
    for b in range(NBUF):
      j = base + b
      _gather(b).wait()
      _idst(j, b).wait()
      sd = _scat(b)
      cd = _cnt(b) if with_cnt else None
      descs.append((sd, cd))
      _isrc((j + NBUF) % q, b).start()
    for b in range(NBUF):
      j = base + b
      sd, cd = descs[b]
      sd.wait()
      if cd is not None:
        cd.wait()
      _idst((j + NBUF) % q, b).start()
      _isrc(j, b).wait()
      _gather(b).start()
    return carry

  lax.fori_loop(0, ng, group, 0)
  for b in range(NBUF):
    _gather(b).wait()
    _idst(b, b).wait()

  def tail(j, carry):
    _isrc(j, 0).start()
    _idst(j, 0).start()
    _isrc(j, 0).wait()
    _idst(j, 0).wait()
    _gather(0).start()
    _gather(0).wait()
    _scat(0).wait()
    if with_cnt:
      _cnt(0).wait()
    return carry

  lax.fori_loop(ng * NBUF, q, tail, 0)


def _tile_quota(c, s):
  w = s * NC + c
  q = QBASE + jnp.where(w < QREM, 1, 0)
  off_e = (w * QBASE + jnp.minimum(w, QREM)) * CH
  return q, off_e


def _seg_sum_cnt_body(feat, edges, z2d, z1d, ones_h, out_p, out_c,
                      acc_sh, cnt_sh, ones_v,
                      isrc0, isrc1, idst0, idst1,
                      rows0, rows1,
                      isem, dsem, gsem, ssem, csem):
  c = lax.axis_index("c")
  s = lax.axis_index("s")
  q, off_e = _tile_quota(c, s)
  base = s * RPS
  pltpu.sync_copy(z2d.at[pl.ds(base, RPS)], acc_sh.at[pl.ds(base, RPS)])

  @pl.when(s == 0)
  def _():
    pltpu.sync_copy(z1d, cnt_sh)

  pltpu.sync_copy(ones_h, ones_v)
  plsc.subcore_barrier()
  _seg_loop(True, q, off_e, feat, edges, acc_sh, cnt_sh, ones_v,
            (isrc0, isrc1), (idst0, idst1),
            (rows0, rows1), isem, dsem, gsem, ssem, csem)
  plsc.subcore_barrier()
  pltpu.sync_copy(acc_sh.at[pl.ds(base, RPS)], out_p.at[c, pl.ds(base, RPS)])

  @pl.when(s == 0)
  def _():
    pltpu.sync_copy(cnt_sh, out_c.at[c])


def _seg_sum_body(feat, edges, z2d, out_p,
                  acc_sh,
                  isrc0, isrc1, idst0, idst1,
                  rows0, rows1,
                  isem, dsem, gsem, ssem):
  c = lax.axis_index("c")
  s = lax.axis_index("s")
  q, off_e = _tile_quota(c, s)
  base = s * RPS
  pltpu.sync_copy(z2d.at[pl.ds(base, RPS)], acc_sh.at[pl.ds(base, RPS)])
  plsc.subcore_barrier()
  _seg_loop(False, q, off_e, feat, edges, acc_sh, None, None,
            (isrc0, isrc1), (idst0, idst1),
            (rows0, rows1), isem, dsem, gsem, ssem, None)
  plsc.subcore_barrier()
  pltpu.sync_copy(acc_sh.at[pl.ds(base, RPS)], out_p.at[c, pl.ds(base, RPS)])


def _make_seg_kernels():
  mesh = plsc.VectorSubcoreMesh(core_axis_name="c", subcore_axis_name="s")
  ring_bufs = (
      [pltpu.VMEM((CH,), jnp.int32) for _ in range(NBUF)] +
      [pltpu.VMEM((1, CH), jnp.int32) for _ in range(NBUF)] +
      [pltpu.VMEM((CH, FDIM), jnp.float32) for _ in range(NBUF)])
  seg_cnt = pl.kernel(
      _seg_sum_cnt_body,
      out_type=(jax.ShapeDtypeStruct((NC, N_PAD, FDIM), jnp.float32),
                jax.ShapeDtypeStruct((NC, N_PAD), jnp.float32)),
      mesh=mesh,
      scratch_types=[
          pltpu.VMEM_SHARED((N_PAD, FDIM), jnp.float32),
          pltpu.VMEM_SHARED((N_PAD,), jnp.float32),
          pltpu.VMEM((CH,), jnp.float32),
      ] + ring_bufs + [
          pltpu.SemaphoreType.DMA((NBUF,)),
          pltpu.SemaphoreType.DMA((NBUF,)),
          pltpu.SemaphoreType.DMA((NBUF,)),
          pltpu.SemaphoreType.DMA((NBUF,)),
          pltpu.SemaphoreType.DMA((NBUF,)),
      ],
      name="sage_seg_sum_cnt",
  )
  seg = pl.kernel(
      _seg_sum_body,
      out_type=jax.ShapeDtypeStruct((NC, N_PAD, FDIM), jnp.float32),
      mesh=mesh,
      scratch_types=[
          pltpu.VMEM_SHARED((N_PAD, FDIM), jnp.float32),
      ] + ring_bufs + [
          pltpu.SemaphoreType.DMA((NBUF,)),
          pltpu.SemaphoreType.DMA((NBUF,)),
          pltpu.SemaphoreType.DMA((NBUF,)),
          pltpu.SemaphoreType.DMA((NBUF,)),
      ],
      name="sage_seg_sum",
  )
  return seg_cnt, seg


BR = 1000


def _dense_body(x_ref, p_ref, c_ref, ws_ref, wn_ref, b_ref, o_ref):
  p = p_ref[0] + p_ref[1]
  cnt = c_ref[0] + c_ref[1]
  agg = p / jnp.maximum(cnt, 1.0)
  acc = jnp.dot(x_ref[...], ws_ref[...], preferred_element_type=jnp.float32)
  acc = acc + jnp.dot(agg, wn_ref[...], preferred_element_type=jnp.float32)
  o_ref[...] = jnp.maximum(acc + b_ref[...], 0.0)


def _dense(x, p, cnt3, ws, wn, b):
  nb = NNODE // BR
  return pl.pallas_call(
      _dense_body,
      grid=(nb,),
      in_specs=[
          pl.BlockSpec((BR, FDIM), lambda i: (i, 0)),
          pl.BlockSpec((NC, BR, FDIM), lambda i: (0, i, 0)),
          pl.BlockSpec((NC, BR, 1), lambda i: (0, i, 0)),
          pl.BlockSpec((FDIM, FDIM), lambda i: (0, 0)),
          pl.BlockSpec((FDIM, FDIM), lambda i: (0, 0)),
          pl.BlockSpec((1, FDIM), lambda i: (0, 0)),
      ],
      out_specs=pl.BlockSpec((BR, FDIM), lambda i: (i, 0)),
      out_shape=jax.ShapeDtypeStruct((NNODE, FDIM), jnp.float32),
  )(x, p, cnt3, ws, wn, b.reshape(1, FDIM))


def kernel(x, edge_index, W_self1, W_neigh1, b1, W_self2, W_neigh2, b2):
  z2d = jnp.zeros((N_PAD, FDIM), jnp.float32)
  z1d = jnp.zeros((N_PAD,), jnp.float32)
  ones_h = jnp.ones((CH,), jnp.float32)

  seg_cnt, seg = _make_seg_kernels()
  p1, cnts = seg_cnt(x, edge_index, z2d, z1d, ones_h)
  cnt3 = cnts.reshape(NC, N_PAD, 1)
  h = _dense(x, p1, cnt3, W_self1, W_neigh1, b1)
  p2 = seg(h, edge_index, z2d)
  return _dense(h, p2, cnt3, W_self2, W_neigh2, b2)


# speedup vs baseline: 3.1596x; 1.0031x over previous
import jax
import jax.numpy as jnp
from jax import lax
from jax.experimental import pallas as pl
from jax.experimental.pallas import tpu as pltpu
from jax.experimental.pallas import tpu_sc as plsc

NC = 2
NS = 16
NW = NC * NS
CH = 128
NBUF = 2
NNODE = 10000
FDIM = 128
N_PAD = 10112
RPS = N_PAD // NS

NCHUNK = -(-320000 // CH)
QBASE = NCHUNK // NW
QREM = NCHUNK % NW


def _seg_loop(with_cnt, q, off_e, feat, edges, acc_sh, cnt_sh, ones_v,
              isrc, idst, rows, isem, dsem, gsem, ssem, csem):
  ng = q // NBUF

  def _isrc(j, b):
    return pltpu.make_async_copy(
        edges.at[0, pl.ds(off_e + j * CH, CH)], isrc[b], isem.at[b])

  def _idst(j, b):
    return pltpu.make_async_copy(
        edges.at[pl.ds(1, 1), pl.ds(off_e + j * CH, CH)], idst[b],
        dsem.at[b])

  def _gather(b):
    return pltpu.make_async_copy(feat.at[isrc[b]], rows[b], gsem.at[b])

  def _scat(b):
    return pltpu.async_copy(rows[b], acc_sh.at[idst[b].at[0]], ssem.at[b],
                            add=True)

  def _cnt(b):
    return pltpu.async_copy(ones_v, cnt_sh.at[idst[b].at[0]], csem.at[b],
                            add=True)

  for b in range(NBUF):
    _isrc(b, b).start()
    _idst(b, b).start()
    _isrc(b, b).wait()
    _gather(b).start()

  def group(t, carry):
    base = t * NBUF
    descs = []
    for b in range(NBUF):
      j = base + b
      _gather(b).wait()
      _idst(j, b).wait()
      sd = _scat(b)
      cd = _cnt(b) if with_cnt else None
      descs.append((sd, cd))
      _isrc((j + NBUF) % q, b).start()
    for b in range(NBUF):
      j = base + b
      sd, cd = descs[b]
      sd.wait()
      if cd is not None:
        cd.wait()
      _idst((j + NBUF) % q, b).start()
      _isrc(j, b).wait()
      _gather(b).start()
    return carry

  lax.fori_loop(0, ng, group, 0)
  for b in range(NBUF):
    _gather(b).wait()
    _idst(b, b).wait()

  def tail(j, carry):
    _isrc(j, 0).start()
    _idst(j, 0).start()
    _isrc(j, 0).wait()
    _idst(j, 0).wait()
    _gather(0).start()
    _gather(0).wait()
    _scat(0).wait()
    if with_cnt:
      _cnt(0).wait()
    return carry

  lax.fori_loop(ng * NBUF, q, tail, 0)


def _tile_quota(c, s):
  w = s * NC + c
  q = QBASE + jnp.where(w < QREM, 1, 0)
  off_e = (w * QBASE + jnp.minimum(w, QREM)) * CH
  return q, off_e


def _seg_sum_cnt_body(feat, edges, z2d, z1d, ones_h, out_p, out_c,
                      acc_sh, cnt_sh, ones_v,
                      isrc0, isrc1, idst0, idst1,
                      rows0, rows1,
                      isem, dsem, gsem, ssem, csem):
  c = lax.axis_index("c")
  s = lax.axis_index("s")
  q, off_e = _tile_quota(c, s)
  base = s * RPS
  pltpu.sync_copy(z2d.at[pl.ds(base, RPS)], acc_sh.at[pl.ds(base, RPS)])

  @pl.when(s == 0)
  def _():
    pltpu.sync_copy(z1d, cnt_sh)

  pltpu.sync_copy(ones_h, ones_v)
  plsc.subcore_barrier()
  _seg_loop(True, q, off_e, feat, edges, acc_sh, cnt_sh, ones_v,
            (isrc0, isrc1), (idst0, idst1),
            (rows0, rows1), isem, dsem, gsem, ssem, csem)
  plsc.subcore_barrier()
  pltpu.sync_copy(acc_sh.at[pl.ds(base, RPS)], out_p.at[c, pl.ds(base, RPS)])

  @pl.when(s == 0)
  def _():
    pltpu.sync_copy(cnt_sh, out_c.at[c])


def _seg_sum_body(feat, edges, z2d, out_p,
                  acc_sh,
                  isrc0, isrc1, idst0, idst1,
                  rows0, rows1,
                  isem, dsem, gsem, ssem):
  c = lax.axis_index("c")
  s = lax.axis_index("s")
  q, off_e = _tile_quota(c, s)
  base = s * RPS
  pltpu.sync_copy(z2d.at[pl.ds(base, RPS)], acc_sh.at[pl.ds(base, RPS)])
  plsc.subcore_barrier()
  _seg_loop(False, q, off_e, feat, edges, acc_sh, None, None,
            (isrc0, isrc1), (idst0, idst1),
            (rows0, rows1), isem, dsem, gsem, ssem, None)
  plsc.subcore_barrier()
  pltpu.sync_copy(acc_sh.at[pl.ds(base, RPS)], out_p.at[c, pl.ds(base, RPS)])


def _make_seg_kernels():
  mesh = plsc.VectorSubcoreMesh(core_axis_name="c", subcore_axis_name="s")
  ring_bufs = (
      [pltpu.VMEM((CH,), jnp.int32) for _ in range(NBUF)] +
      [pltpu.VMEM((1, CH), jnp.int32) for _ in range(NBUF)] +
      [pltpu.VMEM((CH, FDIM), jnp.float32) for _ in range(NBUF)])
  seg_cnt = pl.kernel(
      _seg_sum_cnt_body,
      out_type=(jax.ShapeDtypeStruct((NC, N_PAD, FDIM), jnp.float32),
                jax.ShapeDtypeStruct((NC, N_PAD), jnp.float32)),
      mesh=mesh,
      scratch_types=[
          pltpu.VMEM_SHARED((N_PAD, FDIM), jnp.float32),
          pltpu.VMEM_SHARED((N_PAD,), jnp.float32),
          pltpu.VMEM((CH,), jnp.float32),
      ] + ring_bufs + [
          pltpu.SemaphoreType.DMA((NBUF,)),
          pltpu.SemaphoreType.DMA((NBUF,)),
          pltpu.SemaphoreType.DMA((NBUF,)),
          pltpu.SemaphoreType.DMA((NBUF,)),
          pltpu.SemaphoreType.DMA((NBUF,)),
      ],
      name="sage_seg_sum_cnt",
  )
  seg = pl.kernel(
      _seg_sum_body,
      out_type=jax.ShapeDtypeStruct((NC, N_PAD, FDIM), jnp.float32),
      mesh=mesh,
      scratch_types=[
          pltpu.VMEM_SHARED((N_PAD, FDIM), jnp.float32),
      ] + ring_bufs + [
          pltpu.SemaphoreType.DMA((NBUF,)),
          pltpu.SemaphoreType.DMA((NBUF,)),
          pltpu.SemaphoreType.DMA((NBUF,)),
          pltpu.SemaphoreType.DMA((NBUF,)),
      ],
      name="sage_seg_sum",
  )
  return seg_cnt, seg


BR = 1000


def _pre_body(x_ref, ws_ref, b_ref, o_ref):
  o_ref[...] = (jnp.dot(x_ref[...], ws_ref[...],
                        preferred_element_type=jnp.float32) + b_ref[...])


def _pre(x, ws, b):
  return pl.pallas_call(
      _pre_body,
      grid=(NNODE // BR,),
      in_specs=[
          pl.BlockSpec((BR, FDIM), lambda i: (i, 0)),
          pl.BlockSpec((FDIM, FDIM), lambda i: (0, 0)),
          pl.BlockSpec((1, FDIM), lambda i: (0, 0)),
      ],
      out_specs=pl.BlockSpec((BR, FDIM), lambda i: (i, 0)),
      out_shape=jax.ShapeDtypeStruct((NNODE, FDIM), jnp.float32),
  )(x, ws, b.reshape(1, FDIM))


def _comb_body(pre_ref, p_ref, c_ref, wn_ref, o_ref):
  agg = (p_ref[0] + p_ref[1]) / jnp.maximum(c_ref[0] + c_ref[1], 1.0)
  o_ref[...] = jnp.maximum(
      pre_ref[...] + jnp.dot(agg, wn_ref[...],
                             preferred_element_type=jnp.float32), 0.0)


def _comb(pre, p, cnt3, wn):
  return pl.pallas_call(
      _comb_body,
      grid=(NNODE // BR,),
      in_specs=[
          pl.BlockSpec((BR, FDIM), lambda i: (i, 0)),
          pl.BlockSpec((NC, BR, FDIM), lambda i: (0, i, 0)),
          pl.BlockSpec((NC, BR, 1), lambda i: (0, i, 0)),
          pl.BlockSpec((FDIM, FDIM), lambda i: (0, 0)),
      ],
      out_specs=pl.BlockSpec((BR, FDIM), lambda i: (i, 0)),
      out_shape=jax.ShapeDtypeStruct((NNODE, FDIM), jnp.float32),
  )(pre, p, cnt3, wn)


def kernel(x, edge_index, W_self1, W_neigh1, b1, W_self2, W_neigh2, b2):
  z2d = jnp.zeros((N_PAD, FDIM), jnp.float32)
  z1d = jnp.zeros((N_PAD,), jnp.float32)
  ones_h = jnp.ones((CH,), jnp.float32)

  seg_cnt, seg = _make_seg_kernels()
  p1, cnts = seg_cnt(x, edge_index, z2d, z1d, ones_h)
  pre1 = _pre(x, W_self1, b1)
  cnt3 = cnts.reshape(NC, N_PAD, 1)
  h = _comb(pre1, p1, cnt3, W_neigh1)
  p2 = seg(h, edge_index, z2d)
  pre2 = _pre(h, W_self2, b2)
  return _comb(pre2, p2, cnt3, W_neigh2)


# speedup vs baseline: 3.3782x; 1.0692x over previous
import jax
import jax.numpy as jnp
from jax import lax
from jax.experimental import pallas as pl
from jax.experimental.pallas import tpu as pltpu
from jax.experimental.pallas import tpu_sc as plsc

NC = 2
NS = 16
NW = NC * NS
CH = 128
NBUF_CNT = 2
NBUF_SEG = 3
NNODE = 10000
FDIM = 128
N_PAD = 10112
RPS = N_PAD // NS

NCHUNK = -(-320000 // CH)
QBASE = NCHUNK // NW
QREM = NCHUNK % NW


def _seg_loop(with_cnt, q, off_e, feat, edges, acc_sh, cnt_sh, ones_v,
              isrc, idst, rows, isem, dsem, gsem, ssem, csem):
  nb = len(isrc)
  ng = q // nb

  def _isrc(j, b):
    return pltpu.make_async_copy(
        edges.at[0, pl.ds(off_e + j * CH, CH)], isrc[b], isem.at[b])

  def _idst(j, b):
    return pltpu.make_async_copy(
        edges.at[pl.ds(1, 1), pl.ds(off_e + j * CH, CH)], idst[b],
        dsem.at[b])

  def _gather(b):
    return pltpu.make_async_copy(feat.at[isrc[b]], rows[b], gsem.at[b])

  def _scat(b):
    return pltpu.async_copy(rows[b], acc_sh.at[idst[b].at[0]], ssem.at[b],
                            add=True)

  def _cnt(b):
    return pltpu.async_copy(ones_v, cnt_sh.at[idst[b].at[0]], csem.at[b],
                            add=True)

  for b in range(nb):
    _isrc(b, b).start()
    _idst(b, b).start()
    _isrc(b, b).wait()
    _gather(b).start()

  def group(t, carry):
    base = t * nb
    descs = []
    for b in range(nb):
      j = base + b
      _gather(b).wait()
      _idst(j, b).wait()
      sd = _scat(b)
      cd = _cnt(b) if with_cnt else None
      descs.append((sd, cd))
      _isrc((j + nb) % q, b).start()
    for b in range(nb):
      j = base + b
      sd, cd = descs[b]
      sd.wait()
      if cd is not None:
        cd.wait()
      _idst((j + nb) % q, b).start()
      _isrc(j, b).wait()
      _gather(b).start()
    return carry

  lax.fori_loop(0, ng, group, 0)
  for b in range(nb):
    _gather(b).wait()
    _idst(b, b).wait()

  def tail(j, carry):
    _isrc(j, 0).start()
    _idst(j, 0).start()
    _isrc(j, 0).wait()
    _idst(j, 0).wait()
    _gather(0).start()
    _gather(0).wait()
    _scat(0).wait()
    if with_cnt:
      _cnt(0).wait()
    return carry

  lax.fori_loop(ng * nb, q, tail, 0)


def _tile_quota(c, s):
  w = s * NC + c
  q = QBASE + jnp.where(w < QREM, 1, 0)
  off_e = (w * QBASE + jnp.minimum(w, QREM)) * CH
  return q, off_e


def _seg_sum_cnt_body(feat, edges, z2d, z1d, ones_h, out_p, out_c,
                      acc_sh, cnt_sh, ones_v,
                      isrc0, isrc1, idst0, idst1, rows0, rows1,
                      isem, dsem, gsem, ssem, csem):
  c = lax.axis_index("c")
  s = lax.axis_index("s")
  q, off_e = _tile_quota(c, s)
  base = s * RPS
  pltpu.sync_copy(z2d.at[pl.ds(base, RPS)], acc_sh.at[pl.ds(base, RPS)])

  @pl.when(s == 0)
  def _():
    pltpu.sync_copy(z1d, cnt_sh)

  pltpu.sync_copy(ones_h, ones_v)
  plsc.subcore_barrier()
  _seg_loop(True, q, off_e, feat, edges, acc_sh, cnt_sh, ones_v,
            (isrc0, isrc1), (idst0, idst1),
            (rows0, rows1), isem, dsem, gsem, ssem, csem)
  plsc.subcore_barrier()
  pltpu.sync_copy(acc_sh.at[pl.ds(base, RPS)], out_p.at[c, pl.ds(base, RPS)])

  @pl.when(s == 0)
  def _():
    pltpu.sync_copy(cnt_sh, out_c.at[c])


def _seg_sum_body(feat, edges, z2d, out_p,
                  acc_sh,
                  isrc0, isrc1, isrc2, idst0, idst1, idst2,
                  rows0, rows1, rows2,
                  isem, dsem, gsem, ssem):
  c = lax.axis_index("c")
  s = lax.axis_index("s")
  q, off_e = _tile_quota(c, s)
  base = s * RPS
  pltpu.sync_copy(z2d.at[pl.ds(base, RPS)], acc_sh.at[pl.ds(base, RPS)])
  plsc.subcore_barrier()
  _seg_loop(False, q, off_e, feat, edges, acc_sh, None, None,
            (isrc0, isrc1, isrc2), (idst0, idst1, idst2),
            (rows0, rows1, rows2), isem, dsem, gsem, ssem, None)
  plsc.subcore_barrier()
  pltpu.sync_copy(acc_sh.at[pl.ds(base, RPS)], out_p.at[c, pl.ds(base, RPS)])


def _ring_bufs(nb):
  return (
      [pltpu.VMEM((CH,), jnp.int32) for _ in range(nb)] +
      [pltpu.VMEM((1, CH), jnp.int32) for _ in range(nb)] +
      [pltpu.VMEM((CH, FDIM), jnp.float32) for _ in range(nb)])


def _make_seg_kernels():
  mesh = plsc.VectorSubcoreMesh(core_axis_name="c", subcore_axis_name="s")
  seg_cnt = pl.kernel(
      _seg_sum_cnt_body,
      out_type=(jax.ShapeDtypeStruct((NC, N_PAD, FDIM), jnp.float32),
                jax.ShapeDtypeStruct((NC, N_PAD), jnp.float32)),
      mesh=mesh,
      scratch_types=[
          pltpu.VMEM_SHARED((N_PAD, FDIM), jnp.float32),
          pltpu.VMEM_SHARED((N_PAD,), jnp.float32),
          pltpu.VMEM((CH,), jnp.float32),
      ] + _ring_bufs(NBUF_CNT) + [
          pltpu.SemaphoreType.DMA((NBUF_CNT,)),
          pltpu.SemaphoreType.DMA((NBUF_CNT,)),
          pltpu.SemaphoreType.DMA((NBUF_CNT,)),
          pltpu.SemaphoreType.DMA((NBUF_CNT,)),
          pltpu.SemaphoreType.DMA((NBUF_CNT,)),
      ],
      name="sage_seg_sum_cnt",
  )
  seg = pl.kernel(
      _seg_sum_body,
      out_type=jax.ShapeDtypeStruct((NC, N_PAD, FDIM), jnp.float32),
      mesh=mesh,
      scratch_types=[
          pltpu.VMEM_SHARED((N_PAD, FDIM), jnp.float32),
      ] + _ring_bufs(NBUF_SEG) + [
          pltpu.SemaphoreType.DMA((NBUF_SEG,)),
          pltpu.SemaphoreType.DMA((NBUF_SEG,)),
          pltpu.SemaphoreType.DMA((NBUF_SEG,)),
          pltpu.SemaphoreType.DMA((NBUF_SEG,)),
      ],
      name="sage_seg_sum",
  )
  return seg_cnt, seg


BR = 1000


def _pre_body(x_ref, ws_ref, b_ref, o_ref):
  o_ref[...] = (jnp.dot(x_ref[...], ws_ref[...],
                        preferred_element_type=jnp.float32) + b_ref[...])


def _pre(x, ws, b):
  return pl.pallas_call(
      _pre_body,
      grid=(NNODE // BR,),
      in_specs=[
          pl.BlockSpec((BR, FDIM), lambda i: (i, 0)),
          pl.BlockSpec((FDIM, FDIM), lambda i: (0, 0)),
          pl.BlockSpec((1, FDIM), lambda i: (0, 0)),
      ],
      out_specs=pl.BlockSpec((BR, FDIM), lambda i: (i, 0)),
      out_shape=jax.ShapeDtypeStruct((NNODE, FDIM), jnp.float32),
  )(x, ws, b.reshape(1, FDIM))


def _comb_body(pre_ref, p_ref, c_ref, wn_ref, o_ref):
  agg = (p_ref[0] + p_ref[1]) / jnp.maximum(c_ref[0] + c_ref[1], 1.0)
  o_ref[...] = jnp.maximum(
      pre_ref[...] + jnp.dot(agg, wn_ref[...],
                             preferred_element_type=jnp.float32), 0.0)


def _comb(pre, p, cnt3, wn):
  return pl.pallas_call(
      _comb_body,
      grid=(NNODE // BR,),
      in_specs=[
          pl.BlockSpec((BR, FDIM), lambda i: (i, 0)),
          pl.BlockSpec((NC, BR, FDIM), lambda i: (0, i, 0)),
          pl.BlockSpec((NC, BR, 1), lambda i: (0, i, 0)),
          pl.BlockSpec((FDIM, FDIM), lambda i: (0, 0)),
      ],
      out_specs=pl.BlockSpec((BR, FDIM), lambda i: (i, 0)),
      out_shape=jax.ShapeDtypeStruct((NNODE, FDIM), jnp.float32),
  )(pre, p, cnt3, wn)


def kernel(x, edge_index, W_self1, W_neigh1, b1, W_self2, W_neigh2, b2):
  z2d = jnp.zeros((N_PAD, FDIM), jnp.float32)
  z1d = jnp.zeros((N_PAD,), jnp.float32)
  ones_h = jnp.ones((CH,), jnp.float32)

  seg_cnt, seg = _make_seg_kernels()
  p1, cnts = seg_cnt(x, edge_index, z2d, z1d, ones_h)
  pre1 = _pre(x, W_self1, b1)
  cnt3 = cnts.reshape(NC, N_PAD, 1)
  h = _comb(pre1, p1, cnt3, W_neigh1)
  p2 = seg(h, edge_index, z2d)
  pre2 = _pre(h, W_self2, b2)
  return _comb(pre2, p2, cnt3, W_neigh2)


# speedup vs baseline: 3.5553x; 1.0524x over previous
import jax
import jax.numpy as jnp
from jax import lax
from jax.experimental import pallas as pl
from jax.experimental.pallas import tpu as pltpu
from jax.experimental.pallas import tpu_sc as plsc

NC = 2
NS = 16
NW = NC * NS
CH = 128
NBUF = 3
NNODE = 10000
FDIM = 128
RPS = 632
RPS_LAST = NNODE - (NS - 1) * RPS

NCHUNK = -(-320000 // CH)
QBASE = NCHUNK // NW
QREM = NCHUNK % NW


def _seg_loop(with_cnt, q, off_e, feat, edges, acc_sh, cnt_sh, ones_v,
              isrc, idst, rows, isem, dsem, gsem, ssem, csem):
  nb = len(isrc)
  ng = q // nb

  def _isrc(j, b):
    return pltpu.make_async_copy(
        edges.at[0, pl.ds(off_e + j * CH, CH)], isrc[b], isem.at[b])

  def _idst(j, b):
    return pltpu.make_async_copy(
        edges.at[pl.ds(1, 1), pl.ds(off_e + j * CH, CH)], idst[b],
        dsem.at[b])

  def _gather(b):
    return pltpu.make_async_copy(feat.at[isrc[b]], rows[b], gsem.at[b])

  def _scat(b):
    return pltpu.async_copy(rows[b], acc_sh.at[idst[b].at[0]], ssem.at[b],
                            add=True)

  def _cnt(b):
    return pltpu.async_copy(ones_v, cnt_sh.at[idst[b].at[0]], csem.at[b],
                            add=True)

  for b in range(nb):
    _isrc(b, b).start()
    _idst(b, b).start()
    _isrc(b, b).wait()
    _gather(b).start()

  def group(t, carry):
    base = t * nb
    descs = []
    for b in range(nb):
      j = base + b
      _gather(b).wait()
      _idst(j, b).wait()
      sd = _scat(b)
      cd = _cnt(b) if with_cnt else None
      descs.append((sd, cd))
      _isrc((j + nb) % q, b).start()
    for b in range(nb):
      j = base + b
      sd, cd = descs[b]
      sd.wait()
      if cd is not None:
        cd.wait()
      _idst((j + nb) % q, b).start()
      _isrc(j, b).wait()
      _gather(b).start()
    return carry

  lax.fori_loop(0, ng, group, 0)
  for b in range(nb):
    _gather(b).wait()
    _idst(b, b).wait()

  def tail(j, carry):
    _isrc(j, 0).start()
    _idst(j, 0).start()
    _isrc(j, 0).wait()
    _idst(j, 0).wait()
    _gather(0).start()
    _gather(0).wait()
    _scat(0).wait()
    if with_cnt:
      _cnt(0).wait()
    return carry

  lax.fori_loop(ng * nb, q, tail, 0)


def _tile_quota(c, s):
  w = s * NC + c
  q = QBASE + jnp.where(w < QREM, 1, 0)
  off_e = (w * QBASE + jnp.minimum(w, QREM)) * CH
  return q, off_e


def _slab(s, copy):
  base = s * RPS

  @pl.when(s < NS - 1)
  def _():
    copy(pl.ds(base, RPS))

  @pl.when(s == NS - 1)
  def _():
    copy(pl.ds(base, RPS_LAST))


def _seg_sum_cnt_body(feat, edges, z2d, z1d, ones_h, out_p, out_c,
                      acc_sh, cnt_sh, ones_v,
                      isrc0, isrc1, isrc2, idst0, idst1, idst2,
                      rows0, rows1, rows2,
                      isem, dsem, gsem, ssem, csem):
  c = lax.axis_index("c")
  s = lax.axis_index("s")
  q, off_e = _tile_quota(c, s)
  _slab(s, lambda d: pltpu.sync_copy(z2d.at[d], acc_sh.at[d]))

  @pl.when(s == 0)
  def _():
    pltpu.sync_copy(z1d, cnt_sh)

  pltpu.sync_copy(ones_h, ones_v)
  plsc.subcore_barrier()
  _seg_loop(True, q, off_e, feat, edges, acc_sh, cnt_sh, ones_v,
            (isrc0, isrc1, isrc2), (idst0, idst1, idst2),
            (rows0, rows1, rows2), isem, dsem, gsem, ssem, csem)
  plsc.subcore_barrier()
  _slab(s, lambda d: pltpu.sync_copy(acc_sh.at[d], out_p.at[c, d]))

  @pl.when(s == 0)
  def _():
    pltpu.sync_copy(cnt_sh, out_c.at[c])


def _seg_sum_body(feat, edges, z2d, out_p,
                  acc_sh,
                  isrc0, isrc1, isrc2, idst0, idst1, idst2,
                  rows0, rows1, rows2,
                  isem, dsem, gsem, ssem):
  c = lax.axis_index("c")
  s = lax.axis_index("s")
  q, off_e = _tile_quota(c, s)
  _slab(s, lambda d: pltpu.sync_copy(z2d.at[d], acc_sh.at[d]))
  plsc.subcore_barrier()
  _seg_loop(False, q, off_e, feat, edges, acc_sh, None, None,
            (isrc0, isrc1, isrc2), (idst0, idst1, idst2),
            (rows0, rows1, rows2), isem, dsem, gsem, ssem, None)
  plsc.subcore_barrier()
  _slab(s, lambda d: pltpu.sync_copy(acc_sh.at[d], out_p.at[c, d]))


def _ring_bufs(nb):
  return (
      [pltpu.VMEM((CH,), jnp.int32) for _ in range(nb)] +
      [pltpu.VMEM((1, CH), jnp.int32) for _ in range(nb)] +
      [pltpu.VMEM((CH, FDIM), jnp.float32) for _ in range(nb)])


def _make_seg_kernels():
  mesh = plsc.VectorSubcoreMesh(core_axis_name="c", subcore_axis_name="s")
  seg_cnt = pl.kernel(
      _seg_sum_cnt_body,
      out_type=(jax.ShapeDtypeStruct((NC, NNODE, FDIM), jnp.float32),
                jax.ShapeDtypeStruct((NC, NNODE), jnp.float32)),
      mesh=mesh,
      scratch_types=[
          pltpu.VMEM_SHARED((NNODE, FDIM), jnp.float32),
          pltpu.VMEM_SHARED((NNODE,), jnp.float32),
          pltpu.VMEM((CH,), jnp.float32),
      ] + _ring_bufs(NBUF) + [
          pltpu.SemaphoreType.DMA((NBUF,)),
          pltpu.SemaphoreType.DMA((NBUF,)),
          pltpu.SemaphoreType.DMA((NBUF,)),
          pltpu.SemaphoreType.DMA((NBUF,)),
          pltpu.SemaphoreType.DMA((NBUF,)),
      ],
      name="sage_seg_sum_cnt",
  )
  seg = pl.kernel(
      _seg_sum_body,
      out_type=jax.ShapeDtypeStruct((NC, NNODE, FDIM), jnp.float32),
      mesh=mesh,
      scratch_types=[
          pltpu.VMEM_SHARED((NNODE, FDIM), jnp.float32),
      ] + _ring_bufs(NBUF) + [
          pltpu.SemaphoreType.DMA((NBUF,)),
          pltpu.SemaphoreType.DMA((NBUF,)),
          pltpu.SemaphoreType.DMA((NBUF,)),
          pltpu.SemaphoreType.DMA((NBUF,)),
      ],
      name="sage_seg_sum",
  )
  return seg_cnt, seg


BR = 1000


def _pre_body(x_ref, ws_ref, b_ref, o_ref):
  o_ref[...] = (jnp.dot(x_ref[...], ws_ref[...],
                        preferred_element_type=jnp.float32) + b_ref[...])


def _pre(x, ws, b):
  return pl.pallas_call(
      _pre_body,
      grid=(NNODE // BR,),
      in_specs=[
          pl.BlockSpec((BR, FDIM), lambda i: (i, 0)),
          pl.BlockSpec((FDIM, FDIM), lambda i: (0, 0)),
          pl.BlockSpec((1, FDIM), lambda i: (0, 0)),
      ],
      out_specs=pl.BlockSpec((BR, FDIM), lambda i: (i, 0)),
      out_shape=jax.ShapeDtypeStruct((NNODE, FDIM), jnp.float32),
  )(x, ws, b.reshape(1, FDIM))


def _comb_body(pre_ref, p_ref, c_ref, wn_ref, o_ref):
  agg = (p_ref[0] + p_ref[1]) / jnp.maximum(c_ref[0] + c_ref[1], 1.0)
  o_ref[...] = jnp.maximum(
      pre_ref[...] + jnp.dot(agg, wn_ref[...],
                             preferred_element_type=jnp.float32), 0.0)


def _comb(pre, p, cnt3, wn):
  return pl.pallas_call(
      _comb_body,
      grid=(NNODE // BR,),
      in_specs=[
          pl.BlockSpec((BR, FDIM), lambda i: (i, 0)),
          pl.BlockSpec((NC, BR, FDIM), lambda i: (0, i, 0)),
          pl.BlockSpec((NC, BR, 1), lambda i: (0, i, 0)),
          pl.BlockSpec((FDIM, FDIM), lambda i: (0, 0)),
      ],
      out_specs=pl.BlockSpec((BR, FDIM), lambda i: (i, 0)),
      out_shape=jax.ShapeDtypeStruct((NNODE, FDIM), jnp.float32),
  )(pre, p, cnt3, wn)


def kernel(x, edge_index, W_self1, W_neigh1, b1, W_self2, W_neigh2, b2):
  z2d = jnp.zeros((NNODE, FDIM), jnp.float32)
  z1d = jnp.zeros((NNODE,), jnp.float32)
  ones_h = jnp.ones((CH,), jnp.float32)

  seg_cnt, seg = _make_seg_kernels()
  p1, cnts = seg_cnt(x, edge_index, z2d, z1d, ones_h)
  pre1 = _pre(x, W_self1, b1)
  cnt3 = cnts.reshape(NC, NNODE, 1)
  h = _comb(pre1, p1, cnt3, W_neigh1)
  p2 = seg(h, edge_index, z2d)
  pre2 = _pre(h, W_self2, b2)
  return _comb(pre2, p2, cnt3, W_neigh2)
